# baseline (speedup 1.0000x reference)
import jax
import jax.numpy as jnp
from jax import lax
from jax.experimental import pallas as pl
from jax.experimental.pallas import tpu as pltpu
from jax.experimental.pallas import tpu_sc as plsc

N = 160000
D = 256
NUM_SEGMENTS = 10000

N_SC = 26880
F = N - N_SC

NUM_CORES = 2
NUM_SUBCORES = 16
DH = D // NUM_CORES
RPT = N_SC // NUM_SUBCORES
RB = 80
NIT = RPT // RB
CH0 = F // RB
NCHUNKS = N // RB
ZCH = 80
NZCH = NUM_SEGMENTS // ZCH
KMAX = -(-NZCH // NUM_SUBCORES)

TB = 1024
NTB = F // TB
NTBH = NTB // 2
W = 128
PADS = ((NUM_SEGMENTS - 1) // 8) * 8 + W


def _sc_body(x_hbm, segs_hbm, out_hbm, acc_sh, xb0, xb1, ib0, ib1,
             sem0, sem1, ssem0, ssem1):
    c = lax.axis_index("c")
    s = lax.axis_index("s")
    col0 = c * DH
    zero16 = jnp.zeros((16,), jnp.float32)

    def stage_start(it, xb, ib, sem):
        chunk = CH0 + s * NIT + it
        pltpu.make_async_copy(
            x_hbm.at[pl.ds(chunk * RB, RB), pl.ds(col0, DH)], xb, sem
        ).start()
        pltpu.make_async_copy(segs_hbm.at[chunk], ib, sem).start()

    def stage_wait(it, xb, ib, sem):
        chunk = CH0 + s * NIT + it
        pltpu.make_async_copy(
            x_hbm.at[pl.ds(chunk * RB, RB), pl.ds(col0, DH)], xb, sem
        ).wait()
        pltpu.make_async_copy(segs_hbm.at[chunk], ib, sem).wait()

    def scatter_start(xb, ib, ssem):
        pltpu.async_copy(xb, acc_sh.at[ib.at[0]], ssem, add=True)

    def scatter_wait(xb, ib, ssem):
        pltpu.make_async_copy(xb, acc_sh.at[ib.at[0]], ssem).wait()

    stage_start(0, xb0, ib0, sem0)

    def zero_row(r, carry):
        def zero_lane(j, carry2):
            xb1[r, pl.ds(j * 16, 16)] = zero16
            return carry2
        return lax.fori_loop(0, DH // 16, zero_lane, carry)

    lax.fori_loop(0, ZCH, zero_row, 0)

    def zero_copy(k, carry):
        ch = s + k * NUM_SUBCORES

        @pl.when(ch < NZCH)
        def _():
            pltpu.sync_copy(xb1, acc_sh.at[pl.ds(ch * ZCH, ZCH)])

        return carry

    lax.fori_loop(0, KMAX, zero_copy, 0)
    plsc.subcore_barrier()

    stage_start(1, xb1, ib1, sem1)

    def body(g, carry):
        it0 = 2 * g
        stage_wait(it0, xb0, ib0, sem0)
        scatter_start(xb0, ib0, ssem0)
        stage_wait(it0 + 1, xb1, ib1, sem1)
        scatter_start(xb1, ib1, ssem1)

        @pl.when(it0 + 2 < NIT)
        def _():
            scatter_wait(xb0, ib0, ssem0)
            stage_start(it0 + 2, xb0, ib0, sem0)

        @pl.when(it0 + 3 < NIT)
        def _():
            scatter_wait(xb1, ib1, ssem1)
            stage_start(it0 + 3, xb1, ib1, sem1)

        return carry

    lax.fori_loop(0, NIT // 2, body, 0)
    if NIT % 2 == 1:
        stage_wait(NIT - 1, xb0, ib0, sem0)
        scatter_start(xb0, ib0, ssem0)
        scatter_wait(xb0, ib0, ssem0)
        scatter_wait(xb1, ib1, ssem1)
    else:
        scatter_wait(xb0, ib0, ssem0)
        scatter_wait(xb1, ib1, ssem1)
    plsc.subcore_barrier()

    def wb(k, carry):
        ch = s + k * NUM_SUBCORES

        @pl.when(ch < NZCH)
        def _():
            pltpu.sync_copy(acc_sh.at[pl.ds(ch * ZCH, ZCH)], xb0)
            pltpu.sync_copy(xb0,
                            out_hbm.at[pl.ds(ch * ZCH, ZCH), pl.ds(col0, DH)])

        return carry

    lax.fori_loop(0, KMAX, wb, 0)


def _sc_partial(xs, segs_r):
    f = pl.kernel(
        _sc_body,
        out_type=jax.ShapeDtypeStruct((NUM_SEGMENTS, D), jnp.float32),
        mesh=plsc.VectorSubcoreMesh(core_axis_name="c", subcore_axis_name="s"),
        scratch_types=[
            pltpu.VMEM_SHARED((NUM_SEGMENTS, DH), jnp.float32),
            pltpu.VMEM((RB, DH), jnp.float32),
            pltpu.VMEM((RB, DH), jnp.float32),
            pltpu.VMEM((1, RB), jnp.int32),
            pltpu.VMEM((1, RB), jnp.int32),
            pltpu.SemaphoreType.DMA,
            pltpu.SemaphoreType.DMA,
            pltpu.SemaphoreType.DMA,
            pltpu.SemaphoreType.DMA,
        ],
    )
    return f(xs, segs_r)


def _tc_body(segs_smem, segs_vmem, x_ref, out_ref):
    i = pl.program_id(0)

    @pl.when(i == 0)
    def _():
        out_ref[...] = jnp.zeros(out_ref.shape, out_ref.dtype)

    xbf = x_ref[...].astype(jnp.bfloat16)
    seg_v = segs_vmem[0]
    first = segs_smem[0, 0, 0]

    col_iota = lax.broadcasted_iota(jnp.int32, (1, TB), 1)
    row_iota = lax.broadcasted_iota(jnp.int32, (W, TB), 0)

    def cond(carry):
        r_start, _ = carry
        return r_start < TB

    def body(carry):
        r_start, cur = carry
        base8 = pl.multiple_of((cur // 8) * 8, 8)
        rel = seg_v - base8
        inwin = (col_iota >= r_start) & (rel < W)
        oh = ((row_iota == rel) & inwin).astype(jnp.bfloat16)
        part = lax.dot_general(oh, xbf, (((1,), (0,)), ((), ())),
                               preferred_element_type=jnp.float32)
        out_ref[pl.ds(base8, W), :] = out_ref[pl.ds(base8, W), :] + part
        cnt = jnp.sum(inwin.astype(jnp.int32))
        r_next = r_start + cnt
        nxt = segs_smem[0, 0, jnp.minimum(r_next, TB - 1)]
        return (r_next, nxt)

    lax.while_loop(cond, body, (jnp.int32(0), first))


def _tc_partial(xs, segs3, off):
    return pl.pallas_call(
        _tc_body,
        grid=(NTBH,),
        in_specs=[
            pl.BlockSpec((1, 1, TB), lambda i: (i + off, 0, 0),
                         memory_space=pltpu.SMEM),
            pl.BlockSpec((1, 1, TB), lambda i: (i + off, 0, 0)),
            pl.BlockSpec((TB, D), lambda i: (i + off, 0)),
        ],
        out_specs=pl.BlockSpec((PADS, D), lambda i: (0, 0)),
        out_shape=jax.ShapeDtypeStruct((PADS, D), jnp.float32),
    )(segs3, segs3, xs)


def _merge_body(a_ref, b_ref, c_ref, o_ref):
    o_ref[...] = a_ref[...] + (b_ref[...] + c_ref[...])


def _merge(a, b, c):
    return pl.pallas_call(
        _merge_body,
        grid=(10,),
        in_specs=[
            pl.BlockSpec((NUM_SEGMENTS // 10, D), lambda i: (i, 0)),
            pl.BlockSpec((NUM_SEGMENTS // 10, D), lambda i: (i, 0)),
            pl.BlockSpec((NUM_SEGMENTS // 10, D), lambda i: (i, 0)),
        ],
        out_specs=pl.BlockSpec((NUM_SEGMENTS // 10, D), lambda i: (i, 0)),
        out_shape=jax.ShapeDtypeStruct((NUM_SEGMENTS, D), jnp.float32),
    )(a, b, c)


@jax.jit
def _seg_sum(xs, segs_r, segs3):
    tca = _tc_partial(xs, segs3, 0)
    tcb = _tc_partial(xs, segs3, NTBH)
    sc = _sc_partial(xs, segs_r)
    return _merge(sc, tca, tcb)


def kernel(x, segs):
    xs = jnp.squeeze(x, axis=0)
    segs_r = jnp.reshape(segs, (NCHUNKS, 1, RB))
    segs3 = jnp.reshape(segs[0, :F], (NTB, 1, TB))
    y = _seg_sum(xs, segs_r, segs3)
    return jnp.expand_dims(y, axis=0)


# speedup vs baseline: 1.3813x; 1.3813x over previous
import jax
import jax.numpy as jnp
from jax import lax
from jax.experimental import pallas as pl
from jax.experimental.pallas import tpu as pltpu
from jax.experimental.pallas import tpu_sc as plsc

N = 160000
D = 256
NUM_SEGMENTS = 10000

N_SC = 78080
F = N - N_SC

NUM_CORES = 2
NUM_SUBCORES = 16
DH = D // NUM_CORES
RPT = N_SC // NUM_SUBCORES
RB = 80
NIT = RPT // RB
CH0 = F // RB
NCHUNKS = N // RB
ZCH = 80
NZCH = NUM_SEGMENTS // ZCH
KMAX = -(-NZCH // NUM_SUBCORES)

TB = 1024
NTB = F // TB
NTBH = NTB
W = 128
PADS = ((NUM_SEGMENTS - 1) // 8) * 8 + W


def _sc_body(x_hbm, segs_hbm, out_hbm, acc_sh, xb0, xb1, ib0, ib1,
             sem0, sem1, ssem0, ssem1):
    c = lax.axis_index("c")
    s = lax.axis_index("s")
    col0 = c * DH
    zero16 = jnp.zeros((16,), jnp.float32)

    def stage_start(it, xb, ib, sem):
        chunk = CH0 + s * NIT + it
        pltpu.make_async_copy(
            x_hbm.at[pl.ds(chunk * RB, RB), pl.ds(col0, DH)], xb, sem
        ).start()
        pltpu.make_async_copy(segs_hbm.at[chunk], ib, sem).start()

    def stage_wait(it, xb, ib, sem):
        chunk = CH0 + s * NIT + it
        pltpu.make_async_copy(
            x_hbm.at[pl.ds(chunk * RB, RB), pl.ds(col0, DH)], xb, sem
        ).wait()
        pltpu.make_async_copy(segs_hbm.at[chunk], ib, sem).wait()

    def scatter_start(xb, ib, ssem):
        pltpu.async_copy(xb, acc_sh.at[ib.at[0]], ssem, add=True)

    def scatter_wait(xb, ib, ssem):
        pltpu.make_async_copy(xb, acc_sh.at[ib.at[0]], ssem).wait()

    stage_start(0, xb0, ib0, sem0)

    def zero_row(r, carry):
        def zero_lane(j, carry2):
            xb1[r, pl.ds(j * 16, 16)] = zero16
            return carry2
        return lax.fori_loop(0, DH // 16, zero_lane, carry)

    lax.fori_loop(0, ZCH, zero_row, 0)

    def zero_copy(k, carry):
        ch = s + k * NUM_SUBCORES

        @pl.when(ch < NZCH)
        def _():
            pltpu.sync_copy(xb1, acc_sh.at[pl.ds(ch * ZCH, ZCH)])

        return carry

    lax.fori_loop(0, KMAX, zero_copy, 0)
    plsc.subcore_barrier()

    stage_start(1, xb1, ib1, sem1)

    def body(g, carry):
        it0 = 2 * g
        stage_wait(it0, xb0, ib0, sem0)
        scatter_start(xb0, ib0, ssem0)
        stage_wait(it0 + 1, xb1, ib1, sem1)
        scatter_start(xb1, ib1, ssem1)

        @pl.when(it0 + 2 < NIT)
        def _():
            scatter_wait(xb0, ib0, ssem0)
            stage_start(it0 + 2, xb0, ib0, sem0)

        @pl.when(it0 + 3 < NIT)
        def _():
            scatter_wait(xb1, ib1, ssem1)
            stage_start(it0 + 3, xb1, ib1, sem1)

        return carry

    lax.fori_loop(0, NIT // 2, body, 0)
    if NIT % 2 == 1:
        stage_wait(NIT - 1, xb0, ib0, sem0)
        scatter_start(xb0, ib0, ssem0)
        scatter_wait(xb0, ib0, ssem0)
        scatter_wait(xb1, ib1, ssem1)
    else:
        scatter_wait(xb0, ib0, ssem0)
        scatter_wait(xb1, ib1, ssem1)
    plsc.subcore_barrier()

    def wb(k, carry):
        ch = s + k * NUM_SUBCORES

        @pl.when(ch < NZCH)
        def _():
            pltpu.sync_copy(acc_sh.at[pl.ds(ch * ZCH, ZCH)], xb0)
            pltpu.sync_copy(xb0,
                            out_hbm.at[pl.ds(ch * ZCH, ZCH), pl.ds(col0, DH)])

        return carry

    lax.fori_loop(0, KMAX, wb, 0)


def _sc_partial(xs, segs_r):
    f = pl.kernel(
        _sc_body,
        out_type=jax.ShapeDtypeStruct((NUM_SEGMENTS, D), jnp.float32),
        mesh=plsc.VectorSubcoreMesh(core_axis_name="c", subcore_axis_name="s"),
        scratch_types=[
            pltpu.VMEM_SHARED((NUM_SEGMENTS, DH), jnp.float32),
            pltpu.VMEM((RB, DH), jnp.float32),
            pltpu.VMEM((RB, DH), jnp.float32),
            pltpu.VMEM((1, RB), jnp.int32),
            pltpu.VMEM((1, RB), jnp.int32),
            pltpu.SemaphoreType.DMA,
            pltpu.SemaphoreType.DMA,
            pltpu.SemaphoreType.DMA,
            pltpu.SemaphoreType.DMA,
        ],
    )
    return f(xs, segs_r)


def _tc_body(segs_smem, segs_vmem, x_ref, out_ref):
    i = pl.program_id(0)

    @pl.when(i == 0)
    def _():
        out_ref[...] = jnp.zeros(out_ref.shape, out_ref.dtype)

    xbf = x_ref[...].astype(jnp.bfloat16)
    seg_v = segs_vmem[0]
    first = segs_smem[0, 0, 0]

    col_iota = lax.broadcasted_iota(jnp.int32, (1, TB), 1)
    row_iota = lax.broadcasted_iota(jnp.int32, (W, TB), 0)

    def cond(carry):
        r_start, _ = carry
        return r_start < TB

    def body(carry):
        r_start, cur = carry
        base8 = pl.multiple_of((cur // 8) * 8, 8)
        rel = seg_v - base8
        inwin = (col_iota >= r_start) & (rel < W)
        oh = ((row_iota == rel) & inwin).astype(jnp.bfloat16)
        part = lax.dot_general(oh, xbf, (((1,), (0,)), ((), ())),
                               preferred_element_type=jnp.float32)
        out_ref[pl.ds(base8, W), :] = out_ref[pl.ds(base8, W), :] + part
        cnt = jnp.sum(inwin.astype(jnp.int32))
        r_next = r_start + cnt
        nxt = segs_smem[0, 0, jnp.minimum(r_next, TB - 1)]
        return (r_next, nxt)

    lax.while_loop(cond, body, (jnp.int32(0), first))


def _tc_partial(xs, segs3, off):
    return pl.pallas_call(
        _tc_body,
        grid=(NTBH,),
        in_specs=[
            pl.BlockSpec((1, 1, TB), lambda i: (i + off, 0, 0),
                         memory_space=pltpu.SMEM),
            pl.BlockSpec((1, 1, TB), lambda i: (i + off, 0, 0)),
            pl.BlockSpec((TB, D), lambda i: (i + off, 0)),
        ],
        out_specs=pl.BlockSpec((PADS, D), lambda i: (0, 0)),
        out_shape=jax.ShapeDtypeStruct((PADS, D), jnp.float32),
    )(segs3, segs3, xs)


def _merge_body(a_ref, b_ref, o_ref):
    o_ref[...] = a_ref[...] + b_ref[...]


def _merge(a, b):
    return pl.pallas_call(
        _merge_body,
        grid=(10,),
        in_specs=[
            pl.BlockSpec((NUM_SEGMENTS // 10, D), lambda i: (i, 0)),
            pl.BlockSpec((NUM_SEGMENTS // 10, D), lambda i: (i, 0)),
        ],
        out_specs=pl.BlockSpec((NUM_SEGMENTS // 10, D), lambda i: (i, 0)),
        out_shape=jax.ShapeDtypeStruct((NUM_SEGMENTS, D), jnp.float32),
    )(a, b)


@jax.jit
def _seg_sum(xs, segs_r, segs3):
    tc = _tc_partial(xs, segs3, 0)
    sc = _sc_partial(xs, segs_r)
    return _merge(sc, tc)


def kernel(x, segs):
    xs = jnp.squeeze(x, axis=0)
    segs_r = jnp.reshape(segs, (NCHUNKS, 1, RB))
    segs3 = jnp.reshape(segs[0, :F], (NTB, 1, TB))
    y = _seg_sum(xs, segs_r, segs3)
    return jnp.expand_dims(y, axis=0)


# speedup vs baseline: 1.3858x; 1.0033x over previous
import jax
import jax.numpy as jnp
from jax import lax
from jax.experimental import pallas as pl
from jax.experimental.pallas import tpu as pltpu
from jax.experimental.pallas import tpu_sc as plsc

N = 160000
D = 256
NUM_SEGMENTS = 10000

N_SC = 78080
F = N - N_SC

NUM_CORES = 2
NUM_SUBCORES = 16
DH = D // NUM_CORES
RPT = N_SC // NUM_SUBCORES
RB = 80
NIT = RPT // RB
CH0 = F // RB
NCHUNKS = N // RB
ZCH = 80
NZCH = NUM_SEGMENTS // ZCH
KMAX = -(-NZCH // NUM_SUBCORES)

TB = 1024
NTB = F // TB
NTBH = NTB
W = 96
PADS = ((NUM_SEGMENTS - 1) // 8) * 8 + W


def _sc_body(x_hbm, segs_hbm, out_hbm, acc_sh, xb0, xb1, ib0, ib1,
             sem0, sem1, ssem0, ssem1):
    c = lax.axis_index("c")
    s = lax.axis_index("s")
    col0 = c * DH
    zero16 = jnp.zeros((16,), jnp.float32)

    def stage_start(it, xb, ib, sem):
        chunk = CH0 + s * NIT + it
        pltpu.make_async_copy(
            x_hbm.at[pl.ds(chunk * RB, RB), pl.ds(col0, DH)], xb, sem
        ).start()
        pltpu.make_async_copy(segs_hbm.at[chunk], ib, sem).start()

    def stage_wait(it, xb, ib, sem):
        chunk = CH0 + s * NIT + it
        pltpu.make_async_copy(
            x_hbm.at[pl.ds(chunk * RB, RB), pl.ds(col0, DH)], xb, sem
        ).wait()
        pltpu.make_async_copy(segs_hbm.at[chunk], ib, sem).wait()

    def scatter_start(xb, ib, ssem):
        pltpu.async_copy(xb, acc_sh.at[ib.at[0]], ssem, add=True)

    def scatter_wait(xb, ib, ssem):
        pltpu.make_async_copy(xb, acc_sh.at[ib.at[0]], ssem).wait()

    stage_start(0, xb0, ib0, sem0)

    def zero_row(r, carry):
        def zero_lane(j, carry2):
            xb1[r, pl.ds(j * 16, 16)] = zero16
            return carry2
        return lax.fori_loop(0, DH // 16, zero_lane, carry)

    lax.fori_loop(0, ZCH, zero_row, 0)

    def zero_copy(k, carry):
        ch = s + k * NUM_SUBCORES

        @pl.when(ch < NZCH)
        def _():
            pltpu.sync_copy(xb1, acc_sh.at[pl.ds(ch * ZCH, ZCH)])

        return carry

    lax.fori_loop(0, KMAX, zero_copy, 0)
    plsc.subcore_barrier()

    stage_start(1, xb1, ib1, sem1)

    def body(g, carry):
        it0 = 2 * g
        stage_wait(it0, xb0, ib0, sem0)
        scatter_start(xb0, ib0, ssem0)
        stage_wait(it0 + 1, xb1, ib1, sem1)
        scatter_start(xb1, ib1, ssem1)

        @pl.when(it0 + 2 < NIT)
        def _():
            scatter_wait(xb0, ib0, ssem0)
            stage_start(it0 + 2, xb0, ib0, sem0)

        @pl.when(it0 + 3 < NIT)
        def _():
            scatter_wait(xb1, ib1, ssem1)
            stage_start(it0 + 3, xb1, ib1, sem1)

        return carry

    lax.fori_loop(0, NIT // 2, body, 0)
    if NIT % 2 == 1:
        stage_wait(NIT - 1, xb0, ib0, sem0)
        scatter_start(xb0, ib0, ssem0)
        scatter_wait(xb0, ib0, ssem0)
        scatter_wait(xb1, ib1, ssem1)
    else:
        scatter_wait(xb0, ib0, ssem0)
        scatter_wait(xb1, ib1, ssem1)
    plsc.subcore_barrier()

    def wb(k, carry):
        ch = s + k * NUM_SUBCORES

        @pl.when(ch < NZCH)
        def _():
            pltpu.sync_copy(acc_sh.at[pl.ds(ch * ZCH, ZCH)], xb0)
            pltpu.sync_copy(xb0,
                            out_hbm.at[pl.ds(ch * ZCH, ZCH), pl.ds(col0, DH)])

        return carry

    lax.fori_loop(0, KMAX, wb, 0)


def _sc_partial(xs, segs_r):
    f = pl.kernel(
        _sc_body,
        out_type=jax.ShapeDtypeStruct((NUM_SEGMENTS, D), jnp.float32),
        mesh=plsc.VectorSubcoreMesh(core_axis_name="c", subcore_axis_name="s"),
        scratch_types=[
            pltpu.VMEM_SHARED((NUM_SEGMENTS, DH), jnp.float32),
            pltpu.VMEM((RB, DH), jnp.float32),
            pltpu.VMEM((RB, DH), jnp.float32),
            pltpu.VMEM((1, RB), jnp.int32),
            pltpu.VMEM((1, RB), jnp.int32),
            pltpu.SemaphoreType.DMA,
            pltpu.SemaphoreType.DMA,
            pltpu.SemaphoreType.DMA,
            pltpu.SemaphoreType.DMA,
        ],
    )
    return f(xs, segs_r)


def _tc_body(segs_smem, segs_vmem, x_ref, out_ref):
    i = pl.program_id(0)

    @pl.when(i == 0)
    def _():
        out_ref[...] = jnp.zeros(out_ref.shape, out_ref.dtype)

    xbf = x_ref[...].astype(jnp.bfloat16)
    seg_v = segs_vmem[0]
    first = segs_smem[0, 0, 0]

    col_iota = lax.broadcasted_iota(jnp.int32, (1, TB), 1)
    row_iota = lax.broadcasted_iota(jnp.int32, (W, TB), 0)

    def cond(carry):
        r_start, _ = carry
        return r_start < TB

    def body(carry):
        r_start, cur = carry
        base8 = pl.multiple_of((cur // 8) * 8, 8)
        rel = seg_v - base8
        inwin = (col_iota >= r_start) & (rel < W)
        oh = ((row_iota == rel) & inwin).astype(jnp.bfloat16)
        part = lax.dot_general(oh, xbf, (((1,), (0,)), ((), ())),
                               preferred_element_type=jnp.float32)
        out_ref[pl.ds(base8, W), :] = out_ref[pl.ds(base8, W), :] + part
        cnt = jnp.sum(inwin.astype(jnp.int32))
        r_next = r_start + cnt
        nxt = segs_smem[0, 0, jnp.minimum(r_next, TB - 1)]
        return (r_next, nxt)

    lax.while_loop(cond, body, (jnp.int32(0), first))


def _tc_partial(xs, segs3, off):
    return pl.pallas_call(
        _tc_body,
        grid=(NTBH,),
        in_specs=[
            pl.BlockSpec((1, 1, TB), lambda i: (i + off, 0, 0),
                         memory_space=pltpu.SMEM),
            pl.BlockSpec((1, 1, TB), lambda i: (i + off, 0, 0)),
            pl.BlockSpec((TB, D), lambda i: (i + off, 0)),
        ],
        out_specs=pl.BlockSpec((PADS, D), lambda i: (0, 0)),
        out_shape=jax.ShapeDtypeStruct((PADS, D), jnp.float32),
    )(segs3, segs3, xs)


def _merge_body(a_ref, b_ref, o_ref):
    o_ref[...] = a_ref[...] + b_ref[...]


def _merge(a, b):
    return pl.pallas_call(
        _merge_body,
        grid=(10,),
        in_specs=[
            pl.BlockSpec((NUM_SEGMENTS // 10, D), lambda i: (i, 0)),
            pl.BlockSpec((NUM_SEGMENTS // 10, D), lambda i: (i, 0)),
        ],
        out_specs=pl.BlockSpec((NUM_SEGMENTS // 10, D), lambda i: (i, 0)),
        out_shape=jax.ShapeDtypeStruct((NUM_SEGMENTS, D), jnp.float32),
    )(a, b)


@jax.jit
def _seg_sum(xs, segs_r, segs3):
    tc = _tc_partial(xs, segs3, 0)
    sc = _sc_partial(xs, segs_r)
    return _merge(sc, tc)


def kernel(x, segs):
    xs = jnp.squeeze(x, axis=0)
    segs_r = jnp.reshape(segs, (NCHUNKS, 1, RB))
    segs3 = jnp.reshape(segs[0, :F], (NTB, 1, TB))
    y = _seg_sum(xs, segs_r, segs3)
    return jnp.expand_dims(y, axis=0)


# speedup vs baseline: 1.3943x; 1.0062x over previous
import jax
import jax.numpy as jnp
from jax import lax
from jax.experimental import pallas as pl
from jax.experimental.pallas import tpu as pltpu
from jax.experimental.pallas import tpu_sc as plsc

N = 160000
D = 256
NUM_SEGMENTS = 10000

N_SC = 78080
F = N - N_SC

NUM_CORES = 2
NUM_SUBCORES = 16
DH = D // NUM_CORES
RPT = N_SC // NUM_SUBCORES
RB = 80
NIT = RPT // RB
CH0 = F // RB
NCHUNKS = N // RB
ZCH = 80
NZCH = NUM_SEGMENTS // ZCH
KMAX = -(-NZCH // NUM_SUBCORES)

TB = 1024
NTB = F // TB
NTBH = NTB
W = 96
PADS = ((NUM_SEGMENTS - 1) // 8) * 8 + W


def _sc_body(x_hbm, segs_hbm, out_hbm, acc_sh, xb0, xb1, ib0, ib1,
             sem0, sem1, ssem0, ssem1):
    c = lax.axis_index("c")
    s = lax.axis_index("s")
    col0 = c * DH
    zero16 = jnp.zeros((16,), jnp.float32)

    def stage_start(it, xb, ib, sem):
        chunk = CH0 + s * NIT + it
        pltpu.make_async_copy(
            x_hbm.at[pl.ds(chunk * RB, RB), pl.ds(col0, DH)], xb, sem
        ).start()
        pltpu.make_async_copy(segs_hbm.at[chunk], ib, sem).start()

    def stage_wait(it, xb, ib, sem):
        chunk = CH0 + s * NIT + it
        pltpu.make_async_copy(
            x_hbm.at[pl.ds(chunk * RB, RB), pl.ds(col0, DH)], xb, sem
        ).wait()
        pltpu.make_async_copy(segs_hbm.at[chunk], ib, sem).wait()

    def scatter_start(xb, ib, ssem):
        pltpu.async_copy(xb, acc_sh.at[ib.at[0]], ssem, add=True)

    def scatter_wait(xb, ib, ssem):
        pltpu.make_async_copy(xb, acc_sh.at[ib.at[0]], ssem).wait()

    stage_start(0, xb0, ib0, sem0)

    def zero_row(r, carry):
        def zero_lane(j, carry2):
            xb1[r, pl.ds(j * 16, 16)] = zero16
            return carry2
        return lax.fori_loop(0, DH // 16, zero_lane, carry)

    lax.fori_loop(0, ZCH, zero_row, 0)

    def zero_copy(k, carry):
        ch = s + k * NUM_SUBCORES

        @pl.when(ch < NZCH)
        def _():
            pltpu.sync_copy(xb1, acc_sh.at[pl.ds(ch * ZCH, ZCH)])

        return carry

    lax.fori_loop(0, KMAX, zero_copy, 0)
    plsc.subcore_barrier()

    stage_start(1, xb1, ib1, sem1)

    def body(g, carry):
        it0 = 2 * g
        stage_wait(it0, xb0, ib0, sem0)
        scatter_start(xb0, ib0, ssem0)
        stage_wait(it0 + 1, xb1, ib1, sem1)
        scatter_start(xb1, ib1, ssem1)

        @pl.when(it0 + 2 < NIT)
        def _():
            scatter_wait(xb0, ib0, ssem0)
            stage_start(it0 + 2, xb0, ib0, sem0)

        @pl.when(it0 + 3 < NIT)
        def _():
            scatter_wait(xb1, ib1, ssem1)
            stage_start(it0 + 3, xb1, ib1, sem1)

        return carry

    lax.fori_loop(0, NIT // 2, body, 0)
    if NIT % 2 == 1:
        stage_wait(NIT - 1, xb0, ib0, sem0)
        scatter_start(xb0, ib0, ssem0)
        scatter_wait(xb0, ib0, ssem0)
        scatter_wait(xb1, ib1, ssem1)
    else:
        scatter_wait(xb0, ib0, ssem0)
        scatter_wait(xb1, ib1, ssem1)
    plsc.subcore_barrier()

    def wb(k, carry):
        ch = s + k * NUM_SUBCORES

        @pl.when(ch < NZCH)
        def _():
            pltpu.sync_copy(acc_sh.at[pl.ds(ch * ZCH, ZCH)],
                            out_hbm.at[pl.ds(ch * ZCH, ZCH), pl.ds(col0, DH)])

        return carry

    lax.fori_loop(0, KMAX, wb, 0)


def _sc_partial(xs, segs_r):
    f = pl.kernel(
        _sc_body,
        out_type=jax.ShapeDtypeStruct((NUM_SEGMENTS, D), jnp.float32),
        mesh=plsc.VectorSubcoreMesh(core_axis_name="c", subcore_axis_name="s"),
        scratch_types=[
            pltpu.VMEM_SHARED((NUM_SEGMENTS, DH), jnp.float32),
            pltpu.VMEM((RB, DH), jnp.float32),
            pltpu.VMEM((RB, DH), jnp.float32),
            pltpu.VMEM((1, RB), jnp.int32),
            pltpu.VMEM((1, RB), jnp.int32),
            pltpu.SemaphoreType.DMA,
            pltpu.SemaphoreType.DMA,
            pltpu.SemaphoreType.DMA,
            pltpu.SemaphoreType.DMA,
        ],
    )
    return f(xs, segs_r)


def _tc_body(segs_smem, segs_vmem, x_ref, out_ref):
    i = pl.program_id(0)

    @pl.when(i == 0)
    def _():
        out_ref[...] = jnp.zeros(out_ref.shape, out_ref.dtype)

    xbf = x_ref[...].astype(jnp.bfloat16)
    seg_v = segs_vmem[0]
    first = segs_smem[0, 0, 0]

    col_iota = lax.broadcasted_iota(jnp.int32, (1, TB), 1)
    row_iota = lax.broadcasted_iota(jnp.int32, (W, TB), 0)

    def cond(carry):
        r_start, _ = carry
        return r_start < TB

    def body(carry):
        r_start, cur = carry
        base8 = pl.multiple_of((cur // 8) * 8, 8)
        rel = seg_v - base8
        inwin = (col_iota >= r_start) & (rel < W)
        oh = ((row_iota == rel) & inwin).astype(jnp.bfloat16)
        part = lax.dot_general(oh, xbf, (((1,), (0,)), ((), ())),
                               preferred_element_type=jnp.float32)
        out_ref[pl.ds(base8, W), :] = out_ref[pl.ds(base8, W), :] + part
        cnt = jnp.sum(inwin.astype(jnp.int32))
        r_next = r_start + cnt
        nxt = segs_smem[0, 0, jnp.minimum(r_next, TB - 1)]
        return (r_next, nxt)

    lax.while_loop(cond, body, (jnp.int32(0), first))


def _tc_partial(xs, segs3, off):
    return pl.pallas_call(
        _tc_body,
        grid=(NTBH,),
        in_specs=[
            pl.BlockSpec((1, 1, TB), lambda i: (i + off, 0, 0),
                         memory_space=pltpu.SMEM),
            pl.BlockSpec((1, 1, TB), lambda i: (i + off, 0, 0)),
            pl.BlockSpec((TB, D), lambda i: (i + off, 0)),
        ],
        out_specs=pl.BlockSpec((PADS, D), lambda i: (0, 0)),
        out_shape=jax.ShapeDtypeStruct((PADS, D), jnp.float32),
    )(segs3, segs3, xs)


def _merge_body(a_ref, b_ref, o_ref):
    o_ref[...] = a_ref[...] + b_ref[...]


def _merge(a, b):
    return pl.pallas_call(
        _merge_body,
        grid=(10,),
        in_specs=[
            pl.BlockSpec((NUM_SEGMENTS // 10, D), lambda i: (i, 0)),
            pl.BlockSpec((NUM_SEGMENTS // 10, D), lambda i: (i, 0)),
        ],
        out_specs=pl.BlockSpec((NUM_SEGMENTS // 10, D), lambda i: (i, 0)),
        out_shape=jax.ShapeDtypeStruct((NUM_SEGMENTS, D), jnp.float32),
    )(a, b)


@jax.jit
def _seg_sum(xs, segs_r, segs3):
    tc = _tc_partial(xs, segs3, 0)
    sc = _sc_partial(xs, segs_r)
    return _merge(sc, tc)


def kernel(x, segs):
    xs = jnp.squeeze(x, axis=0)
    segs_r = jnp.reshape(segs, (NCHUNKS, 1, RB))
    segs3 = jnp.reshape(segs[0, :F], (NTB, 1, TB))
    y = _seg_sum(xs, segs_r, segs3)
    return jnp.expand_dims(y, axis=0)
